# Initial kernel scaffold; baseline (speedup 1.0000x reference)
import jax
import jax.numpy as jnp
from jax.experimental import pallas as pl


def kernel(x, target):
    raise NotImplementedError("write your pallas kernel here")


# speedup vs baseline: 2.5431x; 2.5431x over previous
import functools
import math

import jax
import jax.numpy as jnp
from jax import lax
from jax.experimental import pallas as pl
from jax.experimental.pallas import tpu as pltpu
from jax.experimental.pallas import tpu_sc as plsc

_N = 32000
_B = 4096
_PAD = 0
_SMOOTH = 0.1
_CONF = 1.0 - _SMOOTH
_EPS = _SMOOTH / (_N - 2)
_C1 = _CONF * math.log(_CONF) + _SMOOTH * math.log(_EPS)
_CME = _CONF - _EPS

_BR = 512
_BC = 6400
_NW = 32
_PW = _B // _NW


def _sc_gather(x_flat, flat_idx):
    mesh = plsc.VectorSubcoreMesh(core_axis_name="c", subcore_axis_name="s")

    @functools.partial(
        pl.kernel,
        out_type=jax.ShapeDtypeStruct((_B,), jnp.float32),
        mesh=mesh,
        scratch_types=[
            pltpu.VMEM((_PW,), jnp.int32),
            pltpu.VMEM((_PW,), jnp.float32),
            pltpu.SemaphoreType.DMA,
        ],
    )
    def gather_kernel(x_hbm, idx_hbm, out_hbm, idx_v, vals_v, sem):
        wid = lax.axis_index("s") * 2 + lax.axis_index("c")
        base = wid * _PW
        pltpu.sync_copy(idx_hbm.at[pl.ds(base, _PW)], idx_v)
        pltpu.async_copy(x_hbm.at[idx_v], vals_v, sem).wait()
        pltpu.sync_copy(vals_v, out_hbm.at[pl.ds(base, _PW)])

    return gather_kernel(x_flat, flat_idx)


def _tc_loss_body(x_ref, g_ref, t_ref, out_ref, acc_ref):
    c = pl.program_id(1)
    r = pl.program_id(0)
    nc = pl.num_programs(1)
    xb = x_ref[...]

    @pl.when(c == 0)
    def _():
        colid = lax.broadcasted_iota(jnp.int32, (_BR, _BC), 1)
        acc_ref[...] = jnp.sum(jnp.where(colid == 0, 0.0, xb), axis=1)

    @pl.when(c != 0)
    def _():
        acc_ref[...] = acc_ref[...] + jnp.sum(xb, axis=1)

    @pl.when(jnp.logical_and(c == nc - 1, r == 0))
    def _():
        out_ref[0, 0] = 0.0

    @pl.when(c == nc - 1)
    def _():
        m = (t_ref[...] != _PAD).astype(jnp.float32)
        row_loss = m * (_C1 - _CME * g_ref[...] - _EPS * acc_ref[...])
        out_ref[0, 0] += jnp.sum(row_loss)


def _tc_loss(x, g, t32):
    grid = (_B // _BR, _N // _BC)
    return pl.pallas_call(
        _tc_loss_body,
        grid=grid,
        in_specs=[
            pl.BlockSpec((_BR, _BC), lambda r, c: (r, c)),
            pl.BlockSpec((_BR,), lambda r, c: (r,)),
            pl.BlockSpec((_BR,), lambda r, c: (r,)),
        ],
        out_specs=pl.BlockSpec(memory_space=pltpu.SMEM),
        out_shape=jax.ShapeDtypeStruct((1, 1), jnp.float32),
        scratch_shapes=[pltpu.VMEM((_BR,), jnp.float32)],
    )(x, g, t32)


def kernel(x, target):
    t32 = target.astype(jnp.int32)
    flat_idx = jnp.arange(_B, dtype=jnp.int32) * _N + t32
    g = _sc_gather(x.reshape(-1), flat_idx)
    loss = _tc_loss(x, g, t32)
    return loss[0, 0]
